# baseline (speedup 1.0000x reference)
import functools

import jax
import jax.numpy as jnp
from jax import lax
from jax.experimental import pallas as pl
from jax.experimental.pallas import tpu as pltpu
from jax.experimental.pallas import tpu_sc as plsc

B, S, D = 4, 8192, 1024
ROWS = B * S
SCORE_BLK = 512
N_BLK = ROWS // SCORE_BLK


def _scores_body(x_ref, w_ref, o_ref):
    x = x_ref[...]
    w = w_ref[0, :]
    xw = x * w[None, :]
    acc = xw[:, 0:128]
    for k in range(1, 8):
        acc = acc + xw[:, 128 * k:128 * (k + 1)]
    h = 64
    while h >= 1:
        acc = acc[:, :h] + acc[:, h:2 * h]
        h //= 2
    o_ref[...] = acc.reshape(1, 1, SCORE_BLK)


def _scores(flat_inputs, w):
    out = pl.pallas_call(
        _scores_body,
        grid=(N_BLK,),
        in_specs=[
            pl.BlockSpec((SCORE_BLK, D), lambda g: (g, 0)),
            pl.BlockSpec((1, D), lambda g: (0, 0)),
        ],
        out_specs=pl.BlockSpec((1, 1, SCORE_BLK), lambda g: (g, 0, 0)),
        out_shape=jax.ShapeDtypeStruct((N_BLK, 1, SCORE_BLK), jnp.float32),
    )(flat_inputs, w.reshape(1, D))
    return out.reshape(B, S)


_RB = B * (S // 128)
_RG = S // 128


def _sort_body(keys_ref, idx_out_ref):
    keys = keys_ref[...]
    c = lax.broadcasted_iota(jnp.int32, (_RB, 128), 1)
    r = lax.broadcasted_iota(jnp.int32, (_RB, 128), 0)
    i_local = (r & (_RG - 1)) * 128 + c
    idx = (r >> 6) * S + i_local
    size = 2
    while size <= S:
        d = size // 2
        while d >= 1:
            if d < 128:
                bit = (c & d) != 0
                pk = jnp.where(bit, jnp.roll(keys, d, axis=1),
                               jnp.roll(keys, -d, axis=1))
                pi = jnp.where(bit, jnp.roll(idx, d, axis=1),
                               jnp.roll(idx, -d, axis=1))
            else:
                dr = d // 128
                bit = (r & dr) != 0
                pk = jnp.where(bit, jnp.roll(keys, dr, axis=0),
                               jnp.roll(keys, -dr, axis=0))
                pi = jnp.where(bit, jnp.roll(idx, dr, axis=0),
                               jnp.roll(idx, -dr, axis=0))
            keep_min = (~bit) == ((i_local & size) == 0)
            lt = (keys < pk) | ((keys == pk) & (idx < pi))
            take_partner = keep_min ^ lt
            keys = jnp.where(take_partner, pk, keys)
            idx = jnp.where(take_partner, pi, idx)
            d //= 2
        size *= 2
    idx_out_ref[...] = idx


def _argsort(scores):
    packed = pl.pallas_call(
        _sort_body,
        out_shape=jax.ShapeDtypeStruct((_RB, 128), jnp.int32),
    )(scores.reshape(_RB, 128))
    return packed.reshape(B, S)


_NC, _NS = 2, 16
NW = _NC * _NS
RPW = ROWS // NW
CH = 32
NCH = RPW // CH


_NBUF = 3


def _gather_body(table, idx3, out, idx_v, *bufsem):
    bufs = bufsem[:_NBUF]
    sems = bufsem[_NBUF:2 * _NBUF]
    wsems = bufsem[2 * _NBUF:]
    cid = lax.axis_index("c")
    sid = lax.axis_index("s")
    wid = sid * _NC + cid
    pltpu.sync_copy(idx3.at[wid], idx_v)
    base = wid * RPW
    h_g = pltpu.async_copy(table.at[idx_v.at[0]], bufs[0], sems[0])
    h_w = [None] * _NBUF
    for c in range(NCH):
        nxt = c + 1
        h_next = None
        if nxt < NCH:
            if h_w[nxt % _NBUF] is not None:
                h_w[nxt % _NBUF].wait()
            h_next = pltpu.async_copy(
                table.at[idx_v.at[nxt]], bufs[nxt % _NBUF], sems[nxt % _NBUF])
        h_g.wait()
        h_w[c % _NBUF] = pltpu.async_copy(
            bufs[c % _NBUF], out.at[pl.ds(base + c * CH, CH)],
            wsems[c % _NBUF])
        h_g = h_next
    for h in h_w:
        if h is not None:
            h.wait()


_gather = functools.partial(
    pl.kernel,
    mesh=plsc.VectorSubcoreMesh(core_axis_name="c", subcore_axis_name="s"),
    out_type=jax.ShapeDtypeStruct((ROWS, D), jnp.float32),
    scratch_types=(
        [pltpu.VMEM((NCH, CH), jnp.int32)]
        + [pltpu.VMEM((CH, D), jnp.float32)] * _NBUF
        + [pltpu.SemaphoreType.DMA] * (2 * _NBUF)
    ),
)(_gather_body)


def kernel(inputs, w):
    flat = inputs.reshape(ROWS, D)
    scores = jnp.sum(inputs * w, axis=2)
    order = _argsort(scores)
    idx3 = order.reshape(NW, NCH, CH)
    out = _gather(flat, idx3)
    return out.reshape(B, S, D)


# speedup vs baseline: 1.0105x; 1.0105x over previous
import functools

import jax
import jax.numpy as jnp
from jax import lax
from jax.experimental import pallas as pl
from jax.experimental.pallas import tpu as pltpu
from jax.experimental.pallas import tpu_sc as plsc

B, S, D = 4, 8192, 1024
ROWS = B * S
SCORE_BLK = 512
N_BLK = ROWS // SCORE_BLK


def _scores_body(x_ref, w_ref, o_ref):
    x = x_ref[...]
    w = w_ref[0, :]
    xw = x * w[None, :]
    acc = xw[:, 0:128]
    for k in range(1, 8):
        acc = acc + xw[:, 128 * k:128 * (k + 1)]
    h = 64
    while h >= 1:
        acc = acc[:, :h] + acc[:, h:2 * h]
        h //= 2
    o_ref[...] = acc.reshape(1, 1, SCORE_BLK)


def _scores(flat_inputs, w):
    out = pl.pallas_call(
        _scores_body,
        grid=(N_BLK,),
        in_specs=[
            pl.BlockSpec((SCORE_BLK, D), lambda g: (g, 0)),
            pl.BlockSpec((1, D), lambda g: (0, 0)),
        ],
        out_specs=pl.BlockSpec((1, 1, SCORE_BLK), lambda g: (g, 0, 0)),
        out_shape=jax.ShapeDtypeStruct((N_BLK, 1, SCORE_BLK), jnp.float32),
    )(flat_inputs, w.reshape(1, D))
    return out.reshape(B, S)


_RB = B * (S // 128)
_RG = S // 128


def _sort_body(keys_ref, idx_out_ref):
    keys = keys_ref[...]
    c = lax.broadcasted_iota(jnp.int32, (_RB, 128), 1)
    r = lax.broadcasted_iota(jnp.int32, (_RB, 128), 0)
    i_local = (r & (_RG - 1)) * 128 + c
    idx = (r >> 6) * S + i_local
    size = 2
    while size <= S:
        d = size // 2
        while d >= 1:
            if d < 128:
                bit = (c & d) != 0
                pk = jnp.where(bit, jnp.roll(keys, d, axis=1),
                               jnp.roll(keys, -d, axis=1))
                pi = jnp.where(bit, jnp.roll(idx, d, axis=1),
                               jnp.roll(idx, -d, axis=1))
            else:
                dr = d // 128
                bit = (r & dr) != 0
                pk = jnp.where(bit, jnp.roll(keys, dr, axis=0),
                               jnp.roll(keys, -dr, axis=0))
                pi = jnp.where(bit, jnp.roll(idx, dr, axis=0),
                               jnp.roll(idx, -dr, axis=0))
            keep_min = (~bit) == ((i_local & size) == 0)
            lt = (keys < pk) | ((keys == pk) & (idx < pi))
            take_partner = keep_min ^ lt
            keys = jnp.where(take_partner, pk, keys)
            idx = jnp.where(take_partner, pi, idx)
            d //= 2
        size *= 2
    idx_out_ref[...] = idx


def _argsort(scores):
    packed = pl.pallas_call(
        _sort_body,
        out_shape=jax.ShapeDtypeStruct((_RB, 128), jnp.int32),
    )(scores.reshape(_RB, 128))
    return packed.reshape(B, S)


_NC, _NS = 2, 16
NW = _NC * _NS
RPW = ROWS // NW
CH = 16
NCH = RPW // CH


_NBUF = 4


def _gather_body(table, idx3, out, idx_v, *bufsem):
    bufs = bufsem[:_NBUF]
    sems = bufsem[_NBUF:2 * _NBUF]
    wsems = bufsem[2 * _NBUF:]
    cid = lax.axis_index("c")
    sid = lax.axis_index("s")
    wid = sid * _NC + cid
    pltpu.sync_copy(idx3.at[wid], idx_v)
    base = wid * RPW
    h_g = pltpu.async_copy(table.at[idx_v.at[0]], bufs[0], sems[0])
    h_w = [None] * _NBUF
    for c in range(NCH):
        nxt = c + 1
        h_next = None
        if nxt < NCH:
            if h_w[nxt % _NBUF] is not None:
                h_w[nxt % _NBUF].wait()
            h_next = pltpu.async_copy(
                table.at[idx_v.at[nxt]], bufs[nxt % _NBUF], sems[nxt % _NBUF])
        h_g.wait()
        h_w[c % _NBUF] = pltpu.async_copy(
            bufs[c % _NBUF], out.at[pl.ds(base + c * CH, CH)],
            wsems[c % _NBUF])
        h_g = h_next
    for h in h_w:
        if h is not None:
            h.wait()


_gather = functools.partial(
    pl.kernel,
    mesh=plsc.VectorSubcoreMesh(core_axis_name="c", subcore_axis_name="s"),
    out_type=jax.ShapeDtypeStruct((ROWS, D), jnp.float32),
    scratch_types=(
        [pltpu.VMEM((NCH, CH), jnp.int32)]
        + [pltpu.VMEM((CH, D), jnp.float32)] * _NBUF
        + [pltpu.SemaphoreType.DMA] * (2 * _NBUF)
    ),
)(_gather_body)


def kernel(inputs, w):
    flat = inputs.reshape(ROWS, D)
    scores = jnp.sum(inputs * w, axis=2)
    order = _argsort(scores)
    idx3 = order.reshape(NW, NCH, CH)
    out = _gather(flat, idx3)
    return out.reshape(B, S, D)
